# baseline (speedup 1.0000x reference)
import functools

import numpy as np

import jax
import jax.numpy as jnp
from jax.experimental import pallas as pl
from jax.experimental.pallas import tpu as pltpu

_LANE = 128


def _gap_kernel(x_ref, s_ref, o_ref, *, inv_hw):
    xb = x_ref[...].astype(jnp.bfloat16)
    acc = jax.lax.dot_general(
        xb, s_ref[...],
        dimension_numbers=(((1,), (0,)), ((), ())),
        preferred_element_type=jnp.float32,
    )
    o_ref[...] = (acc * inv_hw).astype(o_ref.dtype)


def kernel(x):
    n, c, h, w = x.shape
    hw = h * w
    rows = n * c
    row_len = (hw * _LANE) // int(np.gcd(hw, _LANE))
    groups_per_row = row_len // hw
    rows_o = (rows * hw) // row_len

    x2d = x.reshape(rows_o, row_len)

    stair = (np.arange(row_len)[:, None] // hw
             == np.arange(groups_per_row)[None, :]).astype(np.float32)
    stair = jnp.asarray(stair, dtype=jnp.bfloat16)

    tm = min(128, rows_o)
    grid_m = rows_o // tm

    out = pl.pallas_call(
        functools.partial(_gap_kernel, inv_hw=1.0 / hw),
        out_shape=jax.ShapeDtypeStruct((rows_o, groups_per_row), x.dtype),
        grid=(grid_m,),
        in_specs=[
            pl.BlockSpec((tm, row_len), lambda i: (i, 0)),
            pl.BlockSpec((row_len, groups_per_row), lambda i: (0, 0)),
        ],
        out_specs=pl.BlockSpec((tm, groups_per_row), lambda i: (i, 0)),
        compiler_params=pltpu.CompilerParams(
            dimension_semantics=("parallel",),
        ),
    )(x2d, stair)

    return out.reshape(n, c)


# speedup vs baseline: 35.9757x; 35.9757x over previous
import functools

import jax
import jax.numpy as jnp
from jax.experimental import pallas as pl
from jax.experimental.pallas import tpu as pltpu


def _gap_sum_kernel(x_ref, o_ref, *, hw, inv_hw):
    acc = x_ref[0]
    for k in range(1, hw):
        acc = acc + x_ref[k]
    o_ref[...] = (acc * inv_hw).astype(o_ref.dtype)


def kernel(x):
    n, c, h, w = x.shape
    hw = h * w

    xt = jnp.transpose(x, (2, 3, 0, 1)).reshape(hw, n, c)

    tc = min(256, c)
    grid_c = c // tc

    out = pl.pallas_call(
        functools.partial(_gap_sum_kernel, hw=hw, inv_hw=1.0 / hw),
        out_shape=jax.ShapeDtypeStruct((n, c), x.dtype),
        grid=(grid_c,),
        in_specs=[pl.BlockSpec((hw, n, tc), lambda j: (0, 0, j))],
        out_specs=pl.BlockSpec((n, tc), lambda j: (0, j)),
        compiler_params=pltpu.CompilerParams(
            dimension_semantics=("parallel",),
        ),
    )(xt)

    return out
